# baseline (speedup 1.0000x reference)
import functools
import math

import jax
import jax.numpy as jnp
from jax.experimental import pallas as pl
from jax.experimental.pallas import tpu as pltpu

D_MODEL_K = 4096
MAX_NORM_K = math.sqrt(D_MODEL_K)
CHUNK = 128
NBUF = 16


def _pipeline_kernel(ts_ref, x_hbm, tbl_hbm, o_hbm, buf, emb_ref,
                     in_sems, out_sems, row_sem, *, n_chunks, chunks_per_b,
                     n_batch):
    for b in range(n_batch):
        pltpu.make_async_copy(
            tbl_hbm.at[pl.ds(ts_ref[b], 1), :], emb_ref.at[pl.ds(b, 1), :],
            row_sem,
        ).start()

    def copy_in(i, slot):
        return pltpu.make_async_copy(
            x_hbm.at[pl.ds(i * CHUNK, CHUNK), :],
            buf.at[slot],
            in_sems.at[slot],
        )

    def copy_out(i, slot):
        return pltpu.make_async_copy(
            buf.at[slot],
            o_hbm.at[pl.ds(i * CHUNK, CHUNK), :],
            out_sems.at[slot],
        )

    for s in range(NBUF):
        copy_in(s, s).start()

    for b in range(n_batch):
        pltpu.make_async_copy(
            tbl_hbm.at[pl.ds(ts_ref[b], 1), :], emb_ref.at[pl.ds(b, 1), :],
            row_sem,
        ).wait()
    rows = emb_ref[...]
    norms = jnp.sqrt(jnp.sum(rows * rows, axis=-1, keepdims=True))
    emb_ref[...] = rows * jnp.where(norms > MAX_NORM_K,
                                    MAX_NORM_K / (norms + 1e-7), 1.0)

    def body(i, _):
        slot = jax.lax.rem(i, NBUF)
        copy_in(i, slot).wait()
        b = i // chunks_per_b
        buf[slot] += emb_ref[pl.ds(b, 1), :]
        copy_out(i, slot).start()
        nxt = i + NBUF

        @pl.when(nxt < n_chunks)
        def _():
            copy_out(i, slot).wait()
            copy_in(nxt, slot).start()

        return ()

    jax.lax.fori_loop(0, n_chunks, body, ())

    for s in range(NBUF):
        i = n_chunks - NBUF + s
        copy_out(i, i % NBUF).wait()


def kernel(x, timesteps, table):
    B, S, D = x.shape
    x2 = x.reshape(B * S, D)
    n_chunks = (B * S) // CHUNK
    chunks_per_b = S // CHUNK
    body = functools.partial(_pipeline_kernel, n_chunks=n_chunks,
                             chunks_per_b=chunks_per_b, n_batch=B)
    out = pl.pallas_call(
        body,
        grid_spec=pltpu.PrefetchScalarGridSpec(
            num_scalar_prefetch=1,
            grid=(1,),
            in_specs=[
                pl.BlockSpec(memory_space=pl.ANY),
                pl.BlockSpec(memory_space=pl.ANY),
            ],
            out_specs=pl.BlockSpec(memory_space=pl.ANY),
            scratch_shapes=[
                pltpu.VMEM((NBUF, CHUNK, D), x.dtype),
                pltpu.VMEM((B, D), x.dtype),
                pltpu.SemaphoreType.DMA((NBUF,)),
                pltpu.SemaphoreType.DMA((NBUF,)),
                pltpu.SemaphoreType.DMA,
            ],
        ),
        out_shape=jax.ShapeDtypeStruct(x2.shape, x.dtype),
    )(timesteps, x2, table)
    return out.reshape(B, S, D)


# speedup vs baseline: 1.2169x; 1.2169x over previous
import functools
import math

import jax
import jax.numpy as jnp
from jax.experimental import pallas as pl
from jax.experimental.pallas import tpu as pltpu

D_MODEL_K = 4096
MAX_NORM_K = math.sqrt(D_MODEL_K)
CHUNK = 512
NBUF = 6


def _pipeline_kernel(ts_ref, x_hbm, tbl_hbm, o_hbm, buf, emb_ref,
                     in_sems, out_sems, row_sem, *, n_chunks, chunks_per_b,
                     n_batch):
    for b in range(n_batch):
        pltpu.make_async_copy(
            tbl_hbm.at[pl.ds(ts_ref[b], 1), :], emb_ref.at[pl.ds(b, 1), :],
            row_sem,
        ).start()

    def copy_in(i, slot):
        return pltpu.make_async_copy(
            x_hbm.at[pl.ds(i * CHUNK, CHUNK), :],
            buf.at[slot],
            in_sems.at[slot],
        )

    def copy_out(i, slot):
        return pltpu.make_async_copy(
            buf.at[slot],
            o_hbm.at[pl.ds(i * CHUNK, CHUNK), :],
            out_sems.at[slot],
        )

    for s in range(NBUF):
        copy_in(s, s).start()

    for b in range(n_batch):
        pltpu.make_async_copy(
            tbl_hbm.at[pl.ds(ts_ref[b], 1), :], emb_ref.at[pl.ds(b, 1), :],
            row_sem,
        ).wait()
    rows = emb_ref[...]
    norms = jnp.sqrt(jnp.sum(rows * rows, axis=-1, keepdims=True))
    emb_ref[...] = rows * jnp.where(norms > MAX_NORM_K,
                                    MAX_NORM_K / (norms + 1e-7), 1.0)

    def body(i, _):
        slot = jax.lax.rem(i, NBUF)
        copy_in(i, slot).wait()
        b = i // chunks_per_b
        buf[slot] += emb_ref[pl.ds(b, 1), :]
        copy_out(i, slot).start()
        nxt = i + NBUF

        @pl.when(nxt < n_chunks)
        def _():
            copy_out(i, slot).wait()
            copy_in(nxt, slot).start()

        return ()

    jax.lax.fori_loop(0, n_chunks, body, ())

    for s in range(NBUF):
        i = n_chunks - NBUF + s
        copy_out(i, i % NBUF).wait()


def kernel(x, timesteps, table):
    B, S, D = x.shape
    x2 = x.reshape(B * S, D)
    n_chunks = (B * S) // CHUNK
    chunks_per_b = S // CHUNK
    body = functools.partial(_pipeline_kernel, n_chunks=n_chunks,
                             chunks_per_b=chunks_per_b, n_batch=B)
    out = pl.pallas_call(
        body,
        grid_spec=pltpu.PrefetchScalarGridSpec(
            num_scalar_prefetch=1,
            grid=(1,),
            in_specs=[
                pl.BlockSpec(memory_space=pl.ANY),
                pl.BlockSpec(memory_space=pl.ANY),
            ],
            out_specs=pl.BlockSpec(memory_space=pl.ANY),
            scratch_shapes=[
                pltpu.VMEM((NBUF, CHUNK, D), x.dtype),
                pltpu.VMEM((B, D), x.dtype),
                pltpu.SemaphoreType.DMA((NBUF,)),
                pltpu.SemaphoreType.DMA((NBUF,)),
                pltpu.SemaphoreType.DMA,
            ],
        ),
        out_shape=jax.ShapeDtypeStruct(x2.shape, x.dtype),
    )(timesteps, x2, table)
    return out.reshape(B, S, D)


# speedup vs baseline: 1.2507x; 1.0278x over previous
import functools
import math

import jax
import jax.numpy as jnp
from jax.experimental import pallas as pl
from jax.experimental.pallas import tpu as pltpu

D_MODEL_K = 4096
MAX_NORM_K = math.sqrt(D_MODEL_K)
CHUNK = 1024
NBUF = 3


def _pipeline_kernel(ts_ref, x_hbm, tbl_hbm, o_hbm, buf, emb_ref,
                     in_sems, out_sems, row_sem, *, n_chunks, chunks_per_b,
                     n_batch):
    for b in range(n_batch):
        pltpu.make_async_copy(
            tbl_hbm.at[pl.ds(ts_ref[b], 1), :], emb_ref.at[pl.ds(b, 1), :],
            row_sem,
        ).start()

    def copy_in(i, slot):
        return pltpu.make_async_copy(
            x_hbm.at[pl.ds(i * CHUNK, CHUNK), :],
            buf.at[slot],
            in_sems.at[slot],
        )

    def copy_out(i, slot):
        return pltpu.make_async_copy(
            buf.at[slot],
            o_hbm.at[pl.ds(i * CHUNK, CHUNK), :],
            out_sems.at[slot],
        )

    for s in range(NBUF):
        copy_in(s, s).start()

    for b in range(n_batch):
        pltpu.make_async_copy(
            tbl_hbm.at[pl.ds(ts_ref[b], 1), :], emb_ref.at[pl.ds(b, 1), :],
            row_sem,
        ).wait()
    rows = emb_ref[...]
    norms = jnp.sqrt(jnp.sum(rows * rows, axis=-1, keepdims=True))
    emb_ref[...] = rows * jnp.where(norms > MAX_NORM_K,
                                    MAX_NORM_K / (norms + 1e-7), 1.0)

    def body(i, _):
        slot = jax.lax.rem(i, NBUF)
        copy_in(i, slot).wait()
        b = i // chunks_per_b
        buf[slot] += emb_ref[pl.ds(b, 1), :]
        copy_out(i, slot).start()
        nxt = i + NBUF

        @pl.when(nxt < n_chunks)
        def _():
            copy_out(i, slot).wait()
            copy_in(nxt, slot).start()

        return ()

    jax.lax.fori_loop(0, n_chunks, body, ())

    for s in range(NBUF):
        i = n_chunks - NBUF + s
        copy_out(i, i % NBUF).wait()


def kernel(x, timesteps, table):
    B, S, D = x.shape
    x2 = x.reshape(B * S, D)
    n_chunks = (B * S) // CHUNK
    chunks_per_b = S // CHUNK
    body = functools.partial(_pipeline_kernel, n_chunks=n_chunks,
                             chunks_per_b=chunks_per_b, n_batch=B)
    out = pl.pallas_call(
        body,
        grid_spec=pltpu.PrefetchScalarGridSpec(
            num_scalar_prefetch=1,
            grid=(1,),
            in_specs=[
                pl.BlockSpec(memory_space=pl.ANY),
                pl.BlockSpec(memory_space=pl.ANY),
            ],
            out_specs=pl.BlockSpec(memory_space=pl.ANY),
            scratch_shapes=[
                pltpu.VMEM((NBUF, CHUNK, D), x.dtype),
                pltpu.VMEM((B, D), x.dtype),
                pltpu.SemaphoreType.DMA((NBUF,)),
                pltpu.SemaphoreType.DMA((NBUF,)),
                pltpu.SemaphoreType.DMA,
            ],
        ),
        out_shape=jax.ShapeDtypeStruct(x2.shape, x.dtype),
    )(timesteps, x2, table)
    return out.reshape(B, S, D)
